# baseline (speedup 1.0000x reference)
import functools

import jax
import jax.numpy as jnp
from jax import lax
from jax.experimental import pallas as pl
from jax.experimental.pallas import tpu as pltpu
from jax.experimental.pallas import tpu_sc as plsc

SEQ = 200
EMB = 64
PADDED = 128
BATCH = 1024
VOCAB = 100000
NC = 2
NS = 16
NW = NC * NS
LANES = 16
SCALE = 8.0

STR = SEQ // 8
MAXTR = 7
S_CHUNK = 2
ROWS = S_CHUNK * PADDED


def _sc_embed(idx4, tblv, pos_table):
    mesh = plsc.VectorSubcoreMesh(
        core_axis_name="c", subcore_axis_name="s", num_cores=NC, num_subcores=NS
    )

    @functools.partial(
        pl.kernel,
        mesh=mesh,
        compiler_params=pltpu.CompilerParams(
            use_tc_tiling_on_sc=False, needs_layout_passes=False
        ),
        out_type=jax.ShapeDtypeStruct((SEQ, 8, 8, 8, PADDED), jnp.float32),
        scratch_types=[
            pltpu.VMEM((MAXTR, 8, PADDED), jnp.int32),
            pltpu.VMEM((MAXTR * 8 * PADDED,), jnp.int32),
            pltpu.VMEM((MAXTR * 8, EMB), jnp.float32),
        ]
        + [pltpu.VMEM((ROWS, EMB), jnp.float32) for _ in range(4)]
        + [pltpu.VMEM((S_CHUNK * EMB, PADDED + 1), jnp.float32) for _ in range(2)]
        + [pltpu.SemaphoreType.DMA for _ in range(7)],
    )
    def k(idx_hbm, tok_hbm, pos_hbm, out_hbm, idx_st, idx2_v, pos_v, *rest):
        gbufs = rest[:4]
        obufs = rest[4:6]
        ssem = rest[6]
        gsem = rest[7:11]
        wsem = rest[11:13]
        wid = lax.axis_index("s") * NC + lax.axis_index("c")
        tb = wid // 4
        q = wid % 4
        ntr = jnp.where(q == 0, 7, 6)
        tr0 = jnp.where(q == 0, 0, 7 + (q - 1) * 6)
        nch = ntr * 4

        pstart = jnp.minimum(tr0 * 8, SEQ - MAXTR * 8)
        pltpu.sync_copy(pos_hbm.at[pl.ds(pstart, MAXTR * 8)], pos_v)

        def idx_dma(t, carry):
            pltpu.async_copy(idx_hbm.at[tr0 + t, tb], idx_st.at[t], ssem)
            return carry

        lax.fori_loop(0, ntr, idx_dma, 0)

        def drain_idx(t, carry):
            pltpu.make_async_copy(idx_hbm.at[0, 0], idx_st.at[0], ssem).wait()
            return carry

        lax.fori_loop(0, ntr, drain_idx, 0)

        def idx_flat(t, carry):
            for sl in range(8):
                for h in range(PADDED // LANES):
                    off = t * 1024 + sl * PADDED + h * LANES
                    idx2_v[pl.ds(off, LANES)] = (
                        idx_st[t, sl, pl.ds(h * LANES, LANES)] * 2
                    )
            return carry

        lax.fori_loop(0, ntr, idx_flat, 0)

        def start_gather(kc, b):
            for h in range(2):
                pltpu.async_copy(
                    tok_hbm.at[idx2_v.at[pl.ds(kc * ROWS + h * 128, 128)]],
                    gbufs[b].at[pl.ds(h * 128, 128)],
                    gsem[b],
                )

        def wait_gather(b):
            pltpu.make_async_copy(tok_hbm.at[pl.ds(0, ROWS)], gbufs[b], gsem[b]).wait()

        def start_wb(kc, b):
            s0 = (tr0 + kc // 4) * 8 + (kc % 4) * S_CHUNK
            for ss in range(S_CHUNK):
                for td in range(8):
                    pltpu.async_copy(
                        obufs[b].at[pl.ds(ss * EMB + td * 8, 8), pl.ds(0, PADDED)],
                        out_hbm.at[s0 + ss, td, tb],
                        wsem[b],
                    )

        def wait_wb(b):
            for _ in range(2 * 8):
                pltpu.make_async_copy(
                    obufs[b].at[pl.ds(0, 8), pl.ds(0, PADDED)],
                    out_hbm.at[0, 0, 0],
                    wsem[b],
                ).wait()

        iota = lax.iota(jnp.int32, LANES)

        def compute(kc, gb, ob):
            gbuf = gbufs[gb]
            obuf = obufs[ob]
            s0 = (tr0 + kc // 4) * 8 + (kc % 4) * S_CHUNK
            for ss in range(S_CHUNK):
                prow = s0 + ss - pstart
                pos_j = [
                    pos_v[prow, pl.ds(j * LANES, LANES)]
                    for j in range(EMB // LANES)
                ]
                c_row = [
                    ss * EMB + j * LANES + iota for j in range(EMB // LANES)
                ]

                def b_body(bq, carry):
                    for u in range(4):
                        bv = bq * 4 + u
                        row = ss * PADDED + bv
                        i_b = jnp.full((LANES,), bv, jnp.int32)
                        for j in range(EMB // LANES):
                            v = gbuf[row, pl.ds(j * LANES, LANES)] * SCALE \
                                + pos_j[j]
                            plsc.store_scatter(obuf, [c_row[j], i_b], v)
                    return carry

                lax.fori_loop(0, PADDED // 4, b_body, 0)

        for i in range(3):
            start_gather(i, i)

        def outer(o, carry):
            for phase in range(4):
                kc = 4 * o + phase
                gb = phase
                ob = phase % 2

                @pl.when(kc + 3 < nch)
                def _():
                    start_gather(kc + 3, (phase + 3) % 4)

                wait_gather(gb)

                @pl.when(kc >= 2)
                def _():
                    wait_wb(ob)

                compute(kc, gb, ob)
                start_wb(kc, ob)
            return carry

        lax.fori_loop(0, nch // 4, outer, 0)
        wait_wb(0)
        wait_wb(1)

    return k(idx4, tblv, pos_table)


def kernel(inputs, token_table, pos_table):
    idx4 = (
        inputs.astype(jnp.int32)
        .T.reshape(STR, 8, 8, PADDED)
        .transpose(0, 2, 1, 3)
    )
    tbl128 = jnp.pad(token_table.astype(jnp.float32), ((0, 0), (0, PADDED - EMB)))
    tblv = tbl128.reshape(2 * VOCAB, EMB)
    out5 = _sc_embed(idx4, tblv, pos_table.astype(jnp.float32))
    return jnp.transpose(out5, (2, 4, 0, 1, 3)).reshape(BATCH, SEQ, EMB)


# speedup vs baseline: 1.0543x; 1.0543x over previous
import functools

import jax
import jax.numpy as jnp
from jax import lax
from jax.experimental import pallas as pl
from jax.experimental.pallas import tpu as pltpu
from jax.experimental.pallas import tpu_sc as plsc

SEQ = 200
EMB = 64
PADDED = 128
BATCH = 1024
NC = 2
NS = 16
NW = NC * NS
SEQ_PER_W = BATCH // NW
HALF = SEQ // 2
LANES = 16
SCALE = 8.0
NBUF = 4


def _sc_embed(idx, tblv, pos_table):
    mesh = plsc.VectorSubcoreMesh(
        core_axis_name="c", subcore_axis_name="s", num_cores=NC, num_subcores=NS
    )

    @functools.partial(
        pl.kernel,
        mesh=mesh,
        compiler_params=pltpu.CompilerParams(use_tc_tiling_on_sc=False),
        out_type=jax.ShapeDtypeStruct((BATCH, SEQ, EMB), jnp.float32),
        scratch_types=[
            pltpu.VMEM((SEQ_PER_W, 2, HALF), jnp.int32),
            pltpu.VMEM((SEQ, EMB), jnp.float32),
        ]
        + [pltpu.VMEM((SEQ, EMB), jnp.float32) for _ in range(NBUF)]
        + [pltpu.SemaphoreType.DMA for _ in range(2 * NBUF)],
    )
    def k(idx_hbm, tok_hbm, pos_hbm, out_hbm, idx_v, pos_v, *rest):
        bufs = rest[:NBUF]
        gsem = rest[NBUF : 2 * NBUF]
        wsem = rest[2 * NBUF :]
        wid = lax.axis_index("s") * NC + lax.axis_index("c")
        base = wid * SEQ_PER_W
        pltpu.sync_copy(pos_hbm, pos_v)
        pltpu.sync_copy(idx_hbm.at[pl.ds(base, SEQ_PER_W)], idx_v)

        def start_gather(i, b):
            pltpu.async_copy(
                tok_hbm.at[idx_v.at[i, 0]], bufs[b].at[pl.ds(0, HALF)], gsem[b]
            )
            pltpu.async_copy(
                tok_hbm.at[idx_v.at[i, 1]], bufs[b].at[pl.ds(HALF, HALF)], gsem[b]
            )

        def wait_gather(b):
            pltpu.make_async_copy(out_hbm.at[0], bufs[b], gsem[b]).wait()

        def wait_wb(b):
            pltpu.make_async_copy(bufs[b], out_hbm.at[0], wsem[b]).wait()

        def compute(b):
            buf = bufs[b]

            def row_body(r, carry):
                for rr in range(4):
                    row = r * 4 + rr
                    for j in range(EMB // LANES):
                        sl = pl.ds(j * LANES, LANES)
                        buf[row, sl] = buf[row, sl] * SCALE + pos_v[row, sl]
                return carry

            lax.fori_loop(0, SEQ // 4, row_body, 0)

        for i in range(NBUF - 1):
            start_gather(i, i)

        def outer(o, carry):
            for phase in range(NBUF):
                i = NBUF * o + phase
                b = phase
                nb = (phase + NBUF - 1) % NBUF
                if phase == 0:
                    @pl.when(o >= 1)
                    def _():
                        wait_wb(nb)

                    start_gather(i + NBUF - 1, nb)
                else:
                    @pl.when(o <= SEQ_PER_W // NBUF - 2)
                    def _():
                        wait_wb(nb)
                        start_gather(i + NBUF - 1, nb)

                wait_gather(b)
                compute(b)
                pltpu.async_copy(bufs[b], out_hbm.at[base + i], wsem[b])
            return carry

        lax.fori_loop(0, SEQ_PER_W // NBUF, outer, 0)
        for b in range(NBUF):
            wait_wb(b)

    return k(idx, tblv, pos_table)


def kernel(inputs, token_table, pos_table):
    idx = (inputs.astype(jnp.int32) * 2).reshape(BATCH, 2, HALF)
    tbl128 = jnp.pad(token_table.astype(jnp.float32), ((0, 0), (0, PADDED - EMB)))
    tblv = tbl128.reshape(2 * 100000, EMB)
    return _sc_embed(idx, tblv, pos_table.astype(jnp.float32))
